# Initial kernel scaffold; baseline (speedup 1.0000x reference)
import jax
import jax.numpy as jnp
from jax.experimental import pallas as pl


def kernel(x, gm_w1, gm_b1, gm_w2, gm_b2, qkv_w, ge_w1, ge_b1, ge_w2, ge_b2):
    raise NotImplementedError("write your pallas kernel here")


# speedup vs baseline: 1.4628x; 1.4628x over previous
import jax
import jax.numpy as jnp
from jax import lax
from jax.experimental import pallas as pl
from jax.experimental.pallas import tpu as pltpu

_TILE = 3584


def _pass1_body(x_ref, w1_ref, b1_ref, w2_ref, b2_ref, fac_ref, imp_ref):
    xb = x_ref[0]
    h1 = lax.dot_general(w1_ref[...], xb, (((1,), (0,)), ((), ())),
                         preferred_element_type=jnp.float32)
    h1 = jnp.maximum(h1 + b1_ref[...], 0.0)
    s = lax.dot_general(w2_ref[...], h1, (((1,), (0,)), ((), ())),
                        preferred_element_type=jnp.float32) + b2_ref[...]
    sm = jax.nn.sigmoid(s)
    nh = sm.shape[0]
    c = xb.shape[0]
    fac_ref[0, 0, :] = (0.7 / nh) * jnp.sum(sm, axis=0)
    imp_ref[0, 0, :] = jnp.sum(jnp.abs(xb), axis=0) * (1.0 / c)


def _make_topk_body(num_sel, hw):
    def _topk_body(imp_ref, fac0_ref, fac_ref, vals_ref):
        vals_ref[...] = imp_ref[:, 0, :]
        fac_ref[...] = fac0_ref[...]
        nb = vals_ref.shape[0]
        iota = lax.broadcasted_iota(jnp.int32, (nb, hw), 1)
        add = jnp.float32(0.3 / num_sel)

        def body(i, carry):
            v = vals_ref[...]
            m = jnp.max(v, axis=1, keepdims=True)
            eq = v == m
            first = jnp.min(jnp.where(eq, iota, hw), axis=1, keepdims=True)
            pick = iota == first
            vals_ref[...] = jnp.where(pick, -jnp.inf, v)
            fac_ref[:, 0, :] = fac_ref[:, 0, :] + add * pick.astype(jnp.float32)
            return carry

        lax.fori_loop(0, num_sel, body, 0)
    return _topk_body


def _apply_body(x_ref, fac_ref, o_ref):
    o_ref[0] = x_ref[0] * fac_ref[0, 0, :][None, :]


def kernel(x, gm_w1, gm_b1, gm_w2, gm_b2, qkv_w, ge_w1, ge_b1, ge_w2, ge_b2):
    del qkv_w, ge_w1, ge_b1, ge_w2, ge_b2
    bb, c, h, w = x.shape
    hw = h * w
    hid = gm_w1.shape[0]
    nh = gm_w2.shape[0]
    num_sel = min(max(1, int(hw * 0.01)), 64)

    xf = x.reshape(bb, c, hw)
    b1 = gm_b1.reshape(hid, 1)
    b2 = gm_b2.reshape(nh, 1)

    tile = _TILE if hw % _TILE == 0 else hw
    fac0, imp = pl.pallas_call(
        _pass1_body,
        grid=(bb, hw // tile),
        in_specs=[
            pl.BlockSpec((1, c, tile), lambda b, t: (b, 0, t)),
            pl.BlockSpec((hid, c), lambda b, t: (0, 0)),
            pl.BlockSpec((hid, 1), lambda b, t: (0, 0)),
            pl.BlockSpec((nh, hid), lambda b, t: (0, 0)),
            pl.BlockSpec((nh, 1), lambda b, t: (0, 0)),
        ],
        out_specs=[
            pl.BlockSpec((1, 1, tile), lambda b, t: (b, 0, t)),
            pl.BlockSpec((1, 1, tile), lambda b, t: (b, 0, t)),
        ],
        out_shape=[
            jax.ShapeDtypeStruct((bb, 1, hw), jnp.float32),
            jax.ShapeDtypeStruct((bb, 1, hw), jnp.float32),
        ],
    )(xf, gm_w1, b1, gm_w2, b2)

    fac = pl.pallas_call(
        _make_topk_body(num_sel, hw),
        in_specs=[
            pl.BlockSpec((bb, 1, hw), lambda: (0, 0, 0)),
            pl.BlockSpec((bb, 1, hw), lambda: (0, 0, 0)),
        ],
        out_specs=pl.BlockSpec((bb, 1, hw), lambda: (0, 0, 0)),
        out_shape=jax.ShapeDtypeStruct((bb, 1, hw), jnp.float32),
        scratch_shapes=[pltpu.VMEM((bb, hw), jnp.float32)],
    )(imp, fac0)

    out = pl.pallas_call(
        _apply_body,
        grid=(bb, hw // tile),
        in_specs=[
            pl.BlockSpec((1, c, tile), lambda b, t: (b, 0, t)),
            pl.BlockSpec((1, 1, tile), lambda b, t: (b, 0, t)),
        ],
        out_specs=pl.BlockSpec((1, c, tile), lambda b, t: (b, 0, t)),
        out_shape=jax.ShapeDtypeStruct((bb, c, hw), jnp.float32),
    )(xf, fac)
    return out.reshape(bb, c, h, w)


# speedup vs baseline: 1.4900x; 1.0186x over previous
import jax
import jax.numpy as jnp
from jax import lax
from jax.experimental import pallas as pl
from jax.experimental.pallas import tpu as pltpu

_TILE = 7168


def _pass1_body(x_ref, w1_ref, b1_ref, w2_ref, b2_ref, fac_ref, imp_ref):
    xb = x_ref[0]
    h1 = lax.dot_general(w1_ref[...].astype(jnp.bfloat16), xb.astype(jnp.bfloat16),
                         (((1,), (0,)), ((), ())),
                         preferred_element_type=jnp.float32)
    h1 = jnp.maximum(h1 + b1_ref[...], 0.0)
    s = lax.dot_general(w2_ref[...].astype(jnp.bfloat16), h1.astype(jnp.bfloat16),
                        (((1,), (0,)), ((), ())),
                        preferred_element_type=jnp.float32) + b2_ref[...]
    sm = jax.nn.sigmoid(s)
    nh = sm.shape[0]
    c = xb.shape[0]
    fac_ref[0, 0, :] = (0.7 / nh) * jnp.sum(sm, axis=0)
    imp_ref[0, 0, :] = jnp.sum(jnp.abs(xb), axis=0) * (1.0 / c)


def _make_topk_body(num_sel, hw):
    idx_bits = max(hw - 1, 1).bit_length()

    def _topk_body(imp_ref, fac0_ref, fac_ref):
        nb = imp_ref.shape[0]
        u = lax.bitcast_convert_type(imp_ref[:, 0, :], jnp.uint32)
        inv = jnp.uint32(hw - 1) - lax.broadcasted_iota(jnp.uint32, (nb, hw), 1)

        def cond(tv, ti):
            return (u > tv) | ((u == tv) & (inv >= ti))

        def body(i, carry):
            tv, ti = carry
            bit = jnp.int32(30 + idx_bits) - i
            sh_v = jnp.maximum(bit - idx_bits, 0)
            sh_i = jnp.maximum(bit, 0)
            try_v = tv | jnp.where(bit >= idx_bits,
                                   jnp.uint32(1) << sh_v.astype(jnp.uint32),
                                   jnp.uint32(0))
            try_i = ti | jnp.where(bit < idx_bits,
                                   jnp.uint32(1) << sh_i.astype(jnp.uint32),
                                   jnp.uint32(0))
            cnt = jnp.sum(cond(try_v, try_i).astype(jnp.int32), axis=1,
                          keepdims=True)
            keep = cnt >= num_sel
            return jnp.where(keep, try_v, tv), jnp.where(keep, try_i, ti)

        t0 = jnp.zeros((nb, 1), jnp.uint32)
        tv, ti = lax.fori_loop(0, 31 + idx_bits, body, (t0, t0))
        sel = cond(tv, ti)
        add = jnp.float32(0.3 / num_sel)
        fac_ref[:, 0, :] = fac0_ref[:, 0, :] + add * sel.astype(jnp.float32)

    return _topk_body


def _apply_body(x_ref, fac_ref, o_ref):
    o_ref[0] = x_ref[0] * fac_ref[0]


def kernel(x, gm_w1, gm_b1, gm_w2, gm_b2, qkv_w, ge_w1, ge_b1, ge_w2, ge_b2):
    del qkv_w, ge_w1, ge_b1, ge_w2, ge_b2
    bb, c, h, w = x.shape
    hw = h * w
    hid = gm_w1.shape[0]
    nh = gm_w2.shape[0]
    num_sel = min(max(1, int(hw * 0.01)), 64)
    tile = _TILE if hw % _TILE == 0 else hw

    xf = x.reshape(bb, c, hw)
    b1 = gm_b1.reshape(hid, 1)
    b2 = gm_b2.reshape(nh, 1)

    fac0, imp = pl.pallas_call(
        _pass1_body,
        grid=(bb, hw // tile),
        in_specs=[
            pl.BlockSpec((1, c, tile), lambda b, t: (b, 0, t)),
            pl.BlockSpec((hid, c), lambda b, t: (0, 0)),
            pl.BlockSpec((hid, 1), lambda b, t: (0, 0)),
            pl.BlockSpec((nh, hid), lambda b, t: (0, 0)),
            pl.BlockSpec((nh, 1), lambda b, t: (0, 0)),
        ],
        out_specs=[
            pl.BlockSpec((1, 1, tile), lambda b, t: (b, 0, t)),
            pl.BlockSpec((1, 1, tile), lambda b, t: (b, 0, t)),
        ],
        out_shape=[
            jax.ShapeDtypeStruct((bb, 1, hw), jnp.float32),
            jax.ShapeDtypeStruct((bb, 1, hw), jnp.float32),
        ],
    )(xf, gm_w1, b1, gm_w2, b2)

    fac = pl.pallas_call(
        _make_topk_body(num_sel, hw),
        in_specs=[
            pl.BlockSpec((bb, 1, hw), lambda: (0, 0, 0)),
            pl.BlockSpec((bb, 1, hw), lambda: (0, 0, 0)),
        ],
        out_specs=pl.BlockSpec((bb, 1, hw), lambda: (0, 0, 0)),
        out_shape=jax.ShapeDtypeStruct((bb, 1, hw), jnp.float32),
    )(imp, fac0)

    out = pl.pallas_call(
        _apply_body,
        grid=(bb, hw // tile),
        in_specs=[
            pl.BlockSpec((1, c, tile), lambda b, t: (b, 0, t)),
            pl.BlockSpec((1, 1, tile), lambda b, t: (b, 0, t)),
        ],
        out_specs=pl.BlockSpec((1, c, tile), lambda b, t: (b, 0, t)),
        out_shape=jax.ShapeDtypeStruct((bb, c, hw), jnp.float32),
    )(xf, fac)
    return out.reshape(bb, c, h, w)
